# Initial kernel scaffold; baseline (speedup 1.0000x reference)
import jax
import jax.numpy as jnp
from jax.experimental import pallas as pl


def kernel(self_nodes, neigh_nodes, len_adj_nodes, W, a_self, a_neigh):
    raise NotImplementedError("write your pallas kernel here")


# speedup vs baseline: 2.3762x; 2.3762x over previous
import jax
import jax.numpy as jnp
from jax.experimental import pallas as pl
from jax.experimental.pallas import tpu as pltpu

_N, _K, _D = 10000, 16, 256
_B = 200


def _gat_block(self_ref, neigh_ref, lens_ref, w_ref, a_self_ref,
               a_neigh_ref, out_ref):
    x = self_ref[...]
    nb = neigh_ref[...]
    w = w_ref[...]
    a_s = a_self_ref[...]
    a_n = a_neigh_ref[...]
    lens = lens_ref[...]

    h_self = jnp.dot(x, w, preferred_element_type=jnp.float32)
    h_neigh = jnp.dot(nb, w, preferred_element_type=jnp.float32)

    e_self = jnp.sum(h_self * a_s, axis=1, keepdims=True)
    e_neigh = jnp.sum(h_neigh.reshape(_B, _K, _D) * a_n[None], axis=2)

    e = e_self + e_neigh
    e = jnp.where(e > 0, e, 0.2 * e)

    valid = jax.lax.broadcasted_iota(jnp.int32, (_B, _K), 1) < jnp.maximum(lens, 1)
    e = jnp.where(valid, e, -1e9)

    m = jnp.max(e, axis=1, keepdims=True)
    p = jnp.exp(e - m)
    alpha = p / jnp.sum(p, axis=1, keepdims=True)

    agg = jnp.sum(alpha[:, :, None] * h_neigh.reshape(_B, _K, _D), axis=1)
    z = h_self + agg
    out_ref[...] = jnp.where(z > 0, z, jnp.exp(jnp.minimum(z, 0.0)) - 1.0)


def kernel(self_nodes, neigh_nodes, len_adj_nodes, W, a_self, a_neigh):
    neigh2 = neigh_nodes.reshape(_N * _K, _D)
    lens2 = len_adj_nodes.astype(jnp.int32).reshape(_N, 1)
    a_s2 = a_self.reshape(1, _D)
    a_n2 = a_neigh.reshape(1, _D)

    grid = (_N // _B,)
    return pl.pallas_call(
        _gat_block,
        grid=grid,
        in_specs=[
            pl.BlockSpec((_B, _D), lambda i: (i, 0)),
            pl.BlockSpec((_B * _K, _D), lambda i: (i, 0)),
            pl.BlockSpec((_B, 1), lambda i: (i, 0)),
            pl.BlockSpec((_D, _D), lambda i: (0, 0)),
            pl.BlockSpec((1, _D), lambda i: (0, 0)),
            pl.BlockSpec((1, _D), lambda i: (0, 0)),
        ],
        out_specs=pl.BlockSpec((_B, _D), lambda i: (i, 0)),
        out_shape=jax.ShapeDtypeStruct((_N, _D), jnp.float32),
        compiler_params=pltpu.CompilerParams(
            dimension_semantics=("parallel",),
        ),
    )(self_nodes, neigh2, lens2, W, a_s2, a_n2)


# speedup vs baseline: 2.9460x; 1.2398x over previous
import jax
import jax.numpy as jnp
from jax.experimental import pallas as pl
from jax.experimental.pallas import tpu as pltpu

_N, _K, _D = 10000, 16, 256
_B = 400


def _gat_block(self_ref, neigh_ref, lens_ref, w_ref, a_self_ref,
               a_neigh_ref, out_ref):
    x = self_ref[...]
    nb = neigh_ref[...]
    w = w_ref[...]
    a_s = a_self_ref[...]
    a_n = a_neigh_ref[...]
    lens = lens_ref[...]

    h_self = jnp.dot(x, w, preferred_element_type=jnp.float32)
    h_neigh = jnp.dot(nb, w, preferred_element_type=jnp.float32)

    e_self = jnp.sum(h_self * a_s, axis=1, keepdims=True)
    e_neigh = jnp.sum(h_neigh.reshape(_B, _K, _D) * a_n[None], axis=2)

    e = e_self + e_neigh
    e = jnp.where(e > 0, e, 0.2 * e)

    valid = jax.lax.broadcasted_iota(jnp.int32, (_B, _K), 1) < jnp.maximum(lens, 1)
    e = jnp.where(valid, e, -1e9)

    m = jnp.max(e, axis=1, keepdims=True)
    p = jnp.exp(e - m)
    alpha = p / jnp.sum(p, axis=1, keepdims=True)

    agg = jnp.sum(alpha[:, :, None] * h_neigh.reshape(_B, _K, _D), axis=1)
    z = h_self + agg
    out_ref[...] = jnp.where(z > 0, z, jnp.exp(jnp.minimum(z, 0.0)) - 1.0)


def kernel(self_nodes, neigh_nodes, len_adj_nodes, W, a_self, a_neigh):
    neigh2 = neigh_nodes.reshape(_N * _K, _D)
    lens2 = len_adj_nodes.astype(jnp.int32).reshape(_N, 1)
    a_s2 = a_self.reshape(1, _D)
    a_n2 = a_neigh.reshape(1, _D)

    grid = (_N // _B,)
    return pl.pallas_call(
        _gat_block,
        grid=grid,
        in_specs=[
            pl.BlockSpec((_B, _D), lambda i: (i, 0)),
            pl.BlockSpec((_B * _K, _D), lambda i: (i, 0)),
            pl.BlockSpec((_B, 1), lambda i: (i, 0)),
            pl.BlockSpec((_D, _D), lambda i: (0, 0)),
            pl.BlockSpec((1, _D), lambda i: (0, 0)),
            pl.BlockSpec((1, _D), lambda i: (0, 0)),
        ],
        out_specs=pl.BlockSpec((_B, _D), lambda i: (i, 0)),
        out_shape=jax.ShapeDtypeStruct((_N, _D), jnp.float32),
        compiler_params=pltpu.CompilerParams(
            dimension_semantics=("parallel",),
        ),
    )(self_nodes, neigh2, lens2, W, a_s2, a_n2)


# speedup vs baseline: 3.1514x; 1.0697x over previous
import jax
import jax.numpy as jnp
from jax.experimental import pallas as pl
from jax.experimental.pallas import tpu as pltpu

_N, _K, _D = 10000, 16, 256
_B = 1000


def _gat_block(self_ref, neigh_ref, lens_ref, w_ref, a_self_ref,
               a_neigh_ref, out_ref):
    x = self_ref[...]
    nb = neigh_ref[...]
    w = w_ref[...]
    a_s = a_self_ref[...]
    a_n = a_neigh_ref[...]
    lens = lens_ref[...]

    h_self = jnp.dot(x, w, preferred_element_type=jnp.float32)
    h_neigh = jnp.dot(nb, w, preferred_element_type=jnp.float32)

    e_self = jnp.sum(h_self * a_s, axis=1, keepdims=True)
    e_neigh = jnp.sum(h_neigh.reshape(_B, _K, _D) * a_n[None], axis=2)

    e = e_self + e_neigh
    e = jnp.where(e > 0, e, 0.2 * e)

    valid = jax.lax.broadcasted_iota(jnp.int32, (_B, _K), 1) < jnp.maximum(lens, 1)
    e = jnp.where(valid, e, -1e9)

    m = jnp.max(e, axis=1, keepdims=True)
    p = jnp.exp(e - m)
    alpha = p / jnp.sum(p, axis=1, keepdims=True)

    agg = jnp.sum(alpha[:, :, None] * h_neigh.reshape(_B, _K, _D), axis=1)
    z = h_self + agg
    out_ref[...] = jnp.where(z > 0, z, jnp.exp(jnp.minimum(z, 0.0)) - 1.0)


def kernel(self_nodes, neigh_nodes, len_adj_nodes, W, a_self, a_neigh):
    neigh2 = neigh_nodes.reshape(_N * _K, _D)
    lens2 = len_adj_nodes.astype(jnp.int32).reshape(_N, 1)
    a_s2 = a_self.reshape(1, _D)
    a_n2 = a_neigh.reshape(1, _D)

    grid = (_N // _B,)
    return pl.pallas_call(
        _gat_block,
        grid=grid,
        in_specs=[
            pl.BlockSpec((_B, _D), lambda i: (i, 0)),
            pl.BlockSpec((_B * _K, _D), lambda i: (i, 0)),
            pl.BlockSpec((_B, 1), lambda i: (i, 0)),
            pl.BlockSpec((_D, _D), lambda i: (0, 0)),
            pl.BlockSpec((1, _D), lambda i: (0, 0)),
            pl.BlockSpec((1, _D), lambda i: (0, 0)),
        ],
        out_specs=pl.BlockSpec((_B, _D), lambda i: (i, 0)),
        out_shape=jax.ShapeDtypeStruct((_N, _D), jnp.float32),
        compiler_params=pltpu.CompilerParams(
            dimension_semantics=("parallel",),
        ),
    )(self_nodes, neigh2, lens2, W, a_s2, a_n2)
